# baseline (speedup 1.0000x reference)
import functools

import jax
import jax.numpy as jnp
from jax import lax
from jax.experimental import pallas as pl
from jax.experimental.pallas import tpu as pltpu
from jax.experimental.pallas import tpu_sc as plsc

BATCH = 4096
SEQ = 50
D = 128

B_TOTAL = BATCH * SEQ
NUM_WORKERS = 32
CHUNK = 128
N_CHUNKS = SEQ
NBUF = 2
assert N_CHUNKS % NBUF == 0


def _make_gather():
    mesh = plsc.VectorSubcoreMesh(core_axis_name="c", subcore_axis_name="s")

    @functools.partial(
        pl.kernel,
        mesh=mesh,
        out_type=jax.ShapeDtypeStruct((B_TOTAL, D), jnp.float32),
        scratch_types=[
            pltpu.VMEM((N_CHUNKS, CHUNK), jnp.int32),
            pltpu.VMEM((NBUF, CHUNK, D), jnp.float32),
        ]
        + [pltpu.SemaphoreType.DMA] * (2 * NBUF),
    )
    def gather_kernel(idx_hbm, table_hbm, out_hbm, idx_v, rows_v, *sems):
        gsem = sems[:NBUF]
        ssem = sems[NBUF:]
        wid = lax.axis_index("s") * 2 + lax.axis_index("c")
        bcol = wid * CHUNK
        pltpu.sync_copy(idx_hbm.at[:, pl.ds(bcol, CHUNK)], idx_v)

        def gather_desc(j, b):
            return pltpu.make_async_copy(
                table_hbm.at[idx_v.at[j]], rows_v.at[b], gsem[b])

        def store_desc(j, b):
            return pltpu.make_async_copy(
                rows_v.at[b], out_hbm.at[pl.ds(j * BATCH + bcol, CHUNK)],
                ssem[b])

        for b in range(NBUF):
            gather_desc(b, b).start()

        def body(g, carry):
            j0 = g * NBUF
            for b in range(NBUF):
                gather_desc(j0 + b, b).wait()
                store_desc(j0 + b, b).start()
            for b in range(NBUF):
                store_desc(j0 + b, b).wait()
                gather_desc(j0 + NBUF + b, b).start()
            return carry

        lax.fori_loop(0, N_CHUNKS // NBUF - 1, body, 0)

        j0 = N_CHUNKS - NBUF
        for b in range(NBUF):
            gather_desc(j0 + b, b).wait()
            store_desc(j0 + b, b).start()
        for b in range(NBUF):
            store_desc(j0 + b, b).wait()

    return gather_kernel


_gather = _make_gather()


def kernel(indices, table):
    out = _gather(indices.T, table)
    return out.reshape(SEQ, BATCH, D).transpose(1, 0, 2)


# speedup vs baseline: 1.1066x; 1.1066x over previous
import functools

import jax
import jax.numpy as jnp
from jax import lax
from jax.experimental import pallas as pl
from jax.experimental.pallas import tpu as pltpu
from jax.experimental.pallas import tpu_sc as plsc

BATCH = 4096
SEQ = 50
D = 128

B_TOTAL = BATCH * SEQ
NUM_WORKERS = 32
CHUNK = 128
N_CHUNKS = SEQ
NBUF = 10
assert SEQ % (NBUF // 2) == 0


def _make_gather():
    mesh = plsc.VectorSubcoreMesh(core_axis_name="c", subcore_axis_name="s")

    @functools.partial(
        pl.kernel,
        mesh=mesh,
        out_type=jax.ShapeDtypeStruct((B_TOTAL, D), jnp.float32),
        scratch_types=[
            pltpu.VMEM((N_CHUNKS, CHUNK), jnp.int32),
            pltpu.VMEM((NBUF, CHUNK // 2, D), jnp.float32),
        ]
        + [pltpu.SemaphoreType.DMA] * (2 * NBUF),
    )
    def gather_kernel(idx_hbm, table_hbm, out_hbm, idx_v, rows_v, *sems):
        gsem = sems[:NBUF]
        ssem = sems[NBUF:]
        wid = lax.axis_index("s") * 2 + lax.axis_index("c")
        bcol = wid * CHUNK
        pltpu.sync_copy(idx_hbm.at[:, pl.ds(bcol, CHUNK)], idx_v)

        HALF = CHUNK // 2
        LPG = NBUF // 2

        def gather_desc(l, k, b):
            idx_chunk = idx_v.at[l + k // 2, pl.ds((k % 2) * HALF, HALF)]
            return pltpu.make_async_copy(
                table_hbm.at[idx_chunk], rows_v.at[b], gsem[b])

        def store_desc(l, k, b):
            off = (l + k // 2) * BATCH + bcol + (k % 2) * HALF
            return pltpu.make_async_copy(
                rows_v.at[b], out_hbm.at[pl.ds(off, HALF)], ssem[b])

        for b in range(NBUF):
            gather_desc(0, b, b).start()

        def body(g, carry):
            l0 = g * LPG
            for b in range(NBUF):
                gather_desc(l0, b, b).wait()
                store_desc(l0, b, b).start()
            for b in range(NBUF):
                store_desc(l0, b, b).wait()
                gather_desc(l0 + LPG, b, b).start()
            return carry

        lax.fori_loop(0, SEQ // LPG - 1, body, 0)

        l0 = SEQ - LPG
        for b in range(NBUF):
            gather_desc(l0, b, b).wait()
            store_desc(l0, b, b).start()
        for b in range(NBUF):
            store_desc(l0, b, b).wait()

    return gather_kernel


_gather = _make_gather()


def kernel(indices, table):
    out = _gather(indices.T, table)
    return out.reshape(SEQ, BATCH, D).transpose(1, 0, 2)
